# baseline (speedup 1.0000x reference)
import jax
import jax.numpy as jnp
from jax import lax
from jax.experimental import pallas as pl
from jax.experimental.pallas import tpu as pltpu
from jax.experimental.pallas import tpu_sc as plsc

_NB, _NA, _NC, _NH, _NWG = 8, 3, 80, 64, 64
_AW = (0.05, 0.1, 0.2)
_AH = (0.08, 0.15, 0.3)
_NGT = 200
_NGP = 224
_NWORK = 14
_NCH = 96
_NCELL = _NB * _NA * _NH * _NWG
_NCELLBJ = _NB * _NH * _NWG
_EPS = 1e-7
_DROWS = 128
_NSTEP = _NB * _NH // _DROWS
_GPB = _NGP // _NSTEP


def _iou_anchor(w, h, aw, ah):
    inter = jnp.minimum(w, aw) * jnp.minimum(h, ah)
    union = w * h + aw * ah - inter + 1e-16
    return inter / union


def _best_anchor(w, h):
    i0 = _iou_anchor(w, h, _AW[0], _AH[0])
    i1 = _iou_anchor(w, h, _AW[1], _AH[1])
    i2 = _iou_anchor(w, h, _AW[2], _AH[2])
    best = jnp.where(i1 > i0, 1, 0)
    best = jnp.where(i2 > jnp.maximum(i0, i1), 2, best)
    return best, (i0, i1, i2)


def _sigmoid(x):
    return 1.0 / (1.0 + jnp.exp(-x))


def _sc_assign_body(gts_hbm, res_hbm, gts_v, asn_v):
    wid = lax.axis_index("s") * 2 + lax.axis_index("c")

    @pl.when(wid < _NWORK)
    def _():
        pltpu.sync_copy(gts_hbm, gts_v)
        base = wid * 16

        def fld(f):
            return gts_v[pl.ds(f * _NGP + base, 16)]

        bt, xf, yf, wf, hf = fld(0), fld(2), fld(3), fld(4), fld(5)
        best, _ = _best_anchor(wf, hf)
        bi = bt.astype(jnp.int32)
        gi = (xf * float(_NWG)).astype(jnp.int32)
        gj = (yf * float(_NH)).astype(jnp.int32)
        cell = (bi * _NH + gj) * _NWG + gi
        asn_v[pl.ds(0, 16)] = cell // 8
        asn_v[pl.ds(16, 16)] = cell - (cell // 8) * 8
        asn_v[pl.ds(32, 16)] = best
        pltpu.sync_copy(asn_v.at[pl.ds(0, 16)],
                        res_hbm.at[pl.ds(base, 16)])
        pltpu.sync_copy(asn_v.at[pl.ds(16, 16)],
                        res_hbm.at[pl.ds(_NGP + base, 16)])
        pltpu.sync_copy(asn_v.at[pl.ds(32, 16)],
                        res_hbm.at[pl.ds(2 * _NGP + base, 16)])


def _sparse_math(s_total, gtsr, gtsc, rows):
    f32 = jnp.float32

    def meta(fields):
        b, lab, x, y, w, h = fields
        best, ious = _best_anchor(w, h)
        bi = b.astype(jnp.int32)
        gi = (x * float(_NWG)).astype(jnp.int32)
        gj = (y * float(_NH)).astype(jnp.int32)
        cell = ((bi * _NA + best) * _NH + gj) * _NWG + gi
        anchor_cells = [((bi * _NA + a) * _NH + gj) * _NWG + gi
                        for a in range(_NA)]
        return best, cell, anchor_cells, ious, lab.astype(jnp.int32)

    col = [gtsc[:, k:k + 1] for k in range(6)]
    row = [gtsr[k:k + 1, :] for k in range(6)]
    best_c, cell_c, acells_c, ious_c, lab_c = meta(col)
    best_r, cell_r, acells_r, ious_r, lab_r = meta(row)

    sel = jnp.where(best_c == 0, rows[:, 0:85],
                    jnp.where(best_c == 1, rows[:, 85:170], rows[:, 170:255]))

    gidx_c = lax.broadcasted_iota(jnp.int32, (_NGP, 1), 0)
    gidx_r = lax.broadcasted_iota(jnp.int32, (1, _NGP), 1)
    valid_c = gidx_c < _NGT
    valid_r = gidx_r < _NGT

    eq = cell_c == cell_r
    later = gidx_r > gidx_c
    winner = valid_c & ~jnp.any(eq & later & valid_r, axis=1, keepdims=True)
    eql = (cell_c * _NC + lab_c) == (cell_r * _NC + lab_r)
    earlier = gidx_r < gidx_c
    labrep = valid_c & ~jnp.any(eql & earlier & valid_r, axis=1, keepdims=True)

    k3_c = jnp.concatenate(acells_c, axis=0)
    k3_r = jnp.concatenate(acells_r, axis=1)
    v3_c = jnp.concatenate(
        [(valid_c & ((best_c == a) | (ious_c[a] > 0.5))).astype(jnp.int32)
         for a in range(_NA)], axis=0) != 0
    v3_r = jnp.concatenate(
        [(valid_r & ((best_r == a) | (ious_r[a] > 0.5))).astype(jnp.int32)
         for a in range(_NA)], axis=1) != 0
    i3_c = lax.broadcasted_iota(jnp.int32, (_NA * _NGP, 1), 0)
    i3_r = lax.broadcasted_iota(jnp.int32, (1, _NA * _NGP), 1)
    first3 = v3_c & ~jnp.any((k3_c == k3_r) & (i3_r < i3_c) & v3_r,
                             axis=1, keepdims=True)
    conf3 = jnp.concatenate(
        [rows[:, a * 85 + 4:a * 85 + 5] for a in range(_NA)],
        axis=0)
    p3 = jnp.clip(_sigmoid(conf3), _EPS, 1.0 - _EPS)
    first3f = first3.astype(f32)
    s_nn = jnp.sum(first3f * (-jnp.log(1.0 - p3)))
    n_nn = jnp.sum(first3f)

    winf = winner.astype(f32)
    nm = jnp.sum(winf)
    n_noobj = float(_NCELL) - n_nn
    loss_conf_noobj = (s_total - s_nn) / jnp.maximum(n_noobj, 1.0)

    pobj = jnp.clip(_sigmoid(sel[:, 4:5]), _EPS, 1.0 - _EPS)
    objsum = jnp.sum(winf * (-jnp.log(pobj)))
    loss_conf_obj = objsum / jnp.maximum(nm, 1.0)
    loss_conf = 0.5 * loss_conf_noobj + 1.0 * loss_conf_obj

    xs = _sigmoid(sel[:, 0:1])
    ys = _sigmoid(sel[:, 1:2])
    ws = sel[:, 2:3]
    hs = sel[:, 3:4]
    aw = jnp.where(best_c == 0, _AW[0], jnp.where(best_c == 1, _AW[1], _AW[2]))
    ah = jnp.where(best_c == 0, _AH[0], jnp.where(best_c == 1, _AH[1], _AH[2]))
    tx = col[2] - jnp.floor(col[2])
    ty = col[3] - jnp.floor(col[3])
    reg = ((xs - tx) ** 2 + (ys - ty) ** 2
           + (ws - jnp.log(col[4] / aw + 1e-16)) ** 2
           + (hs - jnp.log(col[5] / ah + 1e-16)) ** 2)
    regsum = jnp.sum(winf * reg)
    loss_reg = regsum / jnp.maximum(nm, 1.0)

    logits = sel[:, 5:85]
    mx = jnp.max(logits, axis=1, keepdims=True)
    e = jnp.exp(logits - mx)
    sm = jnp.clip(e / jnp.sum(e, axis=1, keepdims=True), _EPS, 1.0 - _EPS)
    base_cls = jnp.sum(-jnp.log(1.0 - sm), axis=1, keepdims=True)
    onehot = (lax.broadcasted_iota(jnp.int32, (_NGP, _NC), 1)
              == lab_c).astype(f32)
    lterm = jnp.sum(onehot * (-jnp.log(sm) + jnp.log(1.0 - sm)),
                    axis=1, keepdims=True)
    clssum = jnp.sum(winf * base_cls) + jnp.sum(labrep.astype(f32) * lterm)
    loss_cls = clssum / jnp.maximum(float(_NC) * nm, 1.0)

    return nm * 1.0 * loss_reg + loss_conf + 1.0 * loss_cls * nm


def _tc_main_body(s_ref, dense_ref, gview_ref, gtsr_ref, gtsc_ref, out_ref,
                  raw_scr, stot_scr, sem):
    i = pl.program_id(0)

    def issue(k, _):
        g = i * _GPB + k
        tr = s_ref[g]
        pltpu.make_async_copy(gview_ref.at[pl.ds(tr, 1)],
                              raw_scr.at[pl.ds(g, 1)], sem).start()
        return 0

    lax.fori_loop(0, _GPB, issue, 0)

    s = jnp.float32(0.0)
    for a in range(_NA):
        c = a * (_NC + 5) + 4
        x = dense_ref[:, :, c]
        p = jnp.clip(_sigmoid(x), _EPS, 1.0 - _EPS)
        s = s + jnp.sum(-jnp.log(1.0 - p))

    @pl.when(i == 0)
    def _():
        stot_scr[0, 0] = jnp.float32(0.0)

    stot_scr[0, 0] += s

    @pl.when(i == _NSTEP - 1)
    def _():
        pltpu.make_async_copy(gview_ref.at[pl.ds(0, _NGP)], raw_scr,
                              sem).wait()
        xc = gtsc_ref[:, 2:3]
        yc = gtsc_ref[:, 3:4]
        bc = gtsc_ref[:, 0:1].astype(jnp.int32)
        gi = (xc * float(_NWG)).astype(jnp.int32)
        gj = (yc * float(_NH)).astype(jnp.int32)
        cell = (bc * _NH + gj) * _NWG + gi
        rw = cell - (cell // 8) * 8
        rows = jnp.zeros((_NGP, _NA * (_NC + 5)), jnp.float32)
        for r in range(8):
            m = (rw == r).astype(jnp.float32)
            rows = rows + raw_scr[:, r, :] * m
        total = _sparse_math(stot_scr[0, 0], gtsr_ref[...], gtsc_ref[...],
                             rows)
        out_ref[...] = jnp.broadcast_to(total, (1, 1))


@jax.jit
def kernel(out, gts, size):
    del size
    f32 = jnp.float32
    gts_t = jnp.pad(gts.astype(f32).T, ((0, 0), (0, _NGP - _NGT)))
    gts_t_flat = gts_t.reshape(-1)
    gts_t8 = jnp.pad(gts_t, ((0, 1), (0, 0)))
    gts_p = jnp.pad(gts.astype(f32), ((0, _NGP - _NGT), (0, 1)))

    assign = pl.kernel(
        _sc_assign_body,
        out_type=jax.ShapeDtypeStruct((3 * _NGP,), jnp.int32),
        mesh=plsc.VectorSubcoreMesh(core_axis_name="c", subcore_axis_name="s"),
        compiler_params=pltpu.CompilerParams(needs_layout_passes=False),
        scratch_types=[
            pltpu.VMEM((7 * _NGP,), f32),
            pltpu.VMEM((48,), jnp.int32),
        ],
    )(gts_t_flat)

    nch = _NA * (_NC + 5)
    dview = out.transpose(0, 2, 3, 1).reshape(_NB * _NH, _NWG, nch)
    gview = out.transpose(0, 2, 3, 1).reshape(_NCELLBJ // 8, 8, nch)

    total = pl.pallas_call(
        _tc_main_body,
        grid_spec=pltpu.PrefetchScalarGridSpec(
            num_scalar_prefetch=1,
            grid=(_NSTEP,),
            in_specs=[
                pl.BlockSpec((_DROWS, _NWG, nch), lambda i, s: (i, 0, 0)),
                pl.BlockSpec(memory_space=pl.MemorySpace.ANY),
                pl.BlockSpec((8, _NGP), lambda i, s: (0, 0)),
                pl.BlockSpec((_NGP, 8), lambda i, s: (0, 0)),
            ],
            out_specs=pl.BlockSpec((1, 1), lambda i, s: (0, 0)),
            scratch_shapes=[
                pltpu.VMEM((_NGP, 8, nch), f32),
                pltpu.SMEM((1, 1), f32),
                pltpu.SemaphoreType.DMA,
            ],
        ),
        out_shape=jax.ShapeDtypeStruct((1, 1), f32),
    )(assign, dview, gview, gts_t8, gts_p)
    return total[0, 0]


# speedup vs baseline: 1.0084x; 1.0084x over previous
import jax
import jax.numpy as jnp
from jax import lax
from jax.experimental import pallas as pl
from jax.experimental.pallas import tpu as pltpu
from jax.experimental.pallas import tpu_sc as plsc

_NB, _NA, _NC, _NH, _NWG = 8, 3, 80, 64, 64
_AW = (0.05, 0.1, 0.2)
_AH = (0.08, 0.15, 0.3)
_NGT = 200
_NGP = 224
_NWORK = 14
_NCH = 96
_NCELL = _NB * _NA * _NH * _NWG
_NCELLBJ = _NB * _NH * _NWG
_EPS = 1e-7
_DROWS = 64
_NSTEP = _NB * _NH // _DROWS
_GPB = _NGP // _NSTEP


def _iou_anchor(w, h, aw, ah):
    inter = jnp.minimum(w, aw) * jnp.minimum(h, ah)
    union = w * h + aw * ah - inter + 1e-16
    return inter / union


def _best_anchor(w, h):
    i0 = _iou_anchor(w, h, _AW[0], _AH[0])
    i1 = _iou_anchor(w, h, _AW[1], _AH[1])
    i2 = _iou_anchor(w, h, _AW[2], _AH[2])
    best = jnp.where(i1 > i0, 1, 0)
    best = jnp.where(i2 > jnp.maximum(i0, i1), 2, best)
    return best, (i0, i1, i2)


def _sigmoid(x):
    return 1.0 / (1.0 + jnp.exp(-x))


def _sc_assign_body(gts_hbm, res_hbm, gts_v, asn_v):
    wid = lax.axis_index("s") * 2 + lax.axis_index("c")

    @pl.when(wid < _NWORK)
    def _():
        pltpu.sync_copy(gts_hbm, gts_v)
        base = wid * 16

        def fld(f):
            return gts_v[pl.ds(f * _NGP + base, 16)]

        bt, xf, yf, wf, hf = fld(0), fld(2), fld(3), fld(4), fld(5)
        best, _ = _best_anchor(wf, hf)
        bi = bt.astype(jnp.int32)
        gi = (xf * float(_NWG)).astype(jnp.int32)
        gj = (yf * float(_NH)).astype(jnp.int32)
        cell = (bi * _NH + gj) * _NWG + gi
        asn_v[pl.ds(0, 16)] = cell // 8
        asn_v[pl.ds(16, 16)] = cell - (cell // 8) * 8
        asn_v[pl.ds(32, 16)] = best
        pltpu.sync_copy(asn_v.at[pl.ds(0, 16)],
                        res_hbm.at[pl.ds(base, 16)])
        pltpu.sync_copy(asn_v.at[pl.ds(16, 16)],
                        res_hbm.at[pl.ds(_NGP + base, 16)])
        pltpu.sync_copy(asn_v.at[pl.ds(32, 16)],
                        res_hbm.at[pl.ds(2 * _NGP + base, 16)])


def _sparse_math(s_total, gtsr, gtsc, rows):
    f32 = jnp.float32

    def meta(fields):
        b, lab, x, y, w, h = fields
        best, ious = _best_anchor(w, h)
        bi = b.astype(jnp.int32)
        gi = (x * float(_NWG)).astype(jnp.int32)
        gj = (y * float(_NH)).astype(jnp.int32)
        cell = ((bi * _NA + best) * _NH + gj) * _NWG + gi
        anchor_cells = [((bi * _NA + a) * _NH + gj) * _NWG + gi
                        for a in range(_NA)]
        return best, cell, anchor_cells, ious, lab.astype(jnp.int32)

    col = [gtsc[:, k:k + 1] for k in range(6)]
    row = [gtsr[k:k + 1, :] for k in range(6)]
    best_c, cell_c, acells_c, ious_c, lab_c = meta(col)
    best_r, cell_r, acells_r, ious_r, lab_r = meta(row)

    sel = jnp.where(best_c == 0, rows[:, 0:85],
                    jnp.where(best_c == 1, rows[:, 85:170], rows[:, 170:255]))

    gidx_c = lax.broadcasted_iota(jnp.int32, (_NGP, 1), 0)
    gidx_r = lax.broadcasted_iota(jnp.int32, (1, _NGP), 1)
    valid_c = gidx_c < _NGT
    valid_r = gidx_r < _NGT

    eq = cell_c == cell_r
    later = gidx_r > gidx_c
    winner = valid_c & ~jnp.any(eq & later & valid_r, axis=1, keepdims=True)
    eql = (cell_c * _NC + lab_c) == (cell_r * _NC + lab_r)
    earlier = gidx_r < gidx_c
    labrep = valid_c & ~jnp.any(eql & earlier & valid_r, axis=1, keepdims=True)

    k3_c = jnp.concatenate(acells_c, axis=0)
    k3_r = jnp.concatenate(acells_r, axis=1)
    v3_c = jnp.concatenate(
        [(valid_c & ((best_c == a) | (ious_c[a] > 0.5))).astype(jnp.int32)
         for a in range(_NA)], axis=0) != 0
    v3_r = jnp.concatenate(
        [(valid_r & ((best_r == a) | (ious_r[a] > 0.5))).astype(jnp.int32)
         for a in range(_NA)], axis=1) != 0
    i3_c = lax.broadcasted_iota(jnp.int32, (_NA * _NGP, 1), 0)
    i3_r = lax.broadcasted_iota(jnp.int32, (1, _NA * _NGP), 1)
    first3 = v3_c & ~jnp.any((k3_c == k3_r) & (i3_r < i3_c) & v3_r,
                             axis=1, keepdims=True)
    conf3 = jnp.concatenate(
        [rows[:, a * 85 + 4:a * 85 + 5] for a in range(_NA)],
        axis=0)
    p3 = jnp.clip(_sigmoid(conf3), _EPS, 1.0 - _EPS)
    first3f = first3.astype(f32)
    s_nn = jnp.sum(first3f * (-jnp.log(1.0 - p3)))
    n_nn = jnp.sum(first3f)

    winf = winner.astype(f32)
    nm = jnp.sum(winf)
    n_noobj = float(_NCELL) - n_nn
    loss_conf_noobj = (s_total - s_nn) / jnp.maximum(n_noobj, 1.0)

    pobj = jnp.clip(_sigmoid(sel[:, 4:5]), _EPS, 1.0 - _EPS)
    objsum = jnp.sum(winf * (-jnp.log(pobj)))
    loss_conf_obj = objsum / jnp.maximum(nm, 1.0)
    loss_conf = 0.5 * loss_conf_noobj + 1.0 * loss_conf_obj

    xs = _sigmoid(sel[:, 0:1])
    ys = _sigmoid(sel[:, 1:2])
    ws = sel[:, 2:3]
    hs = sel[:, 3:4]
    aw = jnp.where(best_c == 0, _AW[0], jnp.where(best_c == 1, _AW[1], _AW[2]))
    ah = jnp.where(best_c == 0, _AH[0], jnp.where(best_c == 1, _AH[1], _AH[2]))
    tx = col[2] - jnp.floor(col[2])
    ty = col[3] - jnp.floor(col[3])
    reg = ((xs - tx) ** 2 + (ys - ty) ** 2
           + (ws - jnp.log(col[4] / aw + 1e-16)) ** 2
           + (hs - jnp.log(col[5] / ah + 1e-16)) ** 2)
    regsum = jnp.sum(winf * reg)
    loss_reg = regsum / jnp.maximum(nm, 1.0)

    logits = sel[:, 5:85]
    mx = jnp.max(logits, axis=1, keepdims=True)
    e = jnp.exp(logits - mx)
    sm = jnp.clip(e / jnp.sum(e, axis=1, keepdims=True), _EPS, 1.0 - _EPS)
    base_cls = jnp.sum(-jnp.log(1.0 - sm), axis=1, keepdims=True)
    onehot = (lax.broadcasted_iota(jnp.int32, (_NGP, _NC), 1)
              == lab_c).astype(f32)
    lterm = jnp.sum(onehot * (-jnp.log(sm) + jnp.log(1.0 - sm)),
                    axis=1, keepdims=True)
    clssum = jnp.sum(winf * base_cls) + jnp.sum(labrep.astype(f32) * lterm)
    loss_cls = clssum / jnp.maximum(float(_NC) * nm, 1.0)

    return nm * 1.0 * loss_reg + loss_conf + 1.0 * loss_cls * nm


def _tc_main_body(s_ref, dense_ref, gview_ref, gtsr_ref, gtsc_ref, out_ref,
                  raw_scr, stot_scr, sem):
    i = pl.program_id(0)

    def issue(k, _):
        g = i * _GPB + k
        tr = s_ref[g]
        pltpu.make_async_copy(gview_ref.at[pl.ds(tr, 1)],
                              raw_scr.at[pl.ds(g, 1)], sem).start()
        return 0

    lax.fori_loop(0, _GPB, issue, 0)

    s = jnp.float32(0.0)
    for a in range(_NA):
        c = a * (_NC + 5) + 4
        x = dense_ref[:, :, c]
        p = jnp.clip(_sigmoid(x), _EPS, 1.0 - _EPS)
        s = s + jnp.sum(-jnp.log(1.0 - p))

    @pl.when(i == 0)
    def _():
        stot_scr[0, 0] = jnp.float32(0.0)

    stot_scr[0, 0] += s

    @pl.when(i == _NSTEP - 1)
    def _():
        pltpu.make_async_copy(gview_ref.at[pl.ds(0, _NGP)], raw_scr,
                              sem).wait()
        xc = gtsc_ref[:, 2:3]
        yc = gtsc_ref[:, 3:4]
        bc = gtsc_ref[:, 0:1].astype(jnp.int32)
        gi = (xc * float(_NWG)).astype(jnp.int32)
        gj = (yc * float(_NH)).astype(jnp.int32)
        cell = (bc * _NH + gj) * _NWG + gi
        rw = cell - (cell // 8) * 8
        rows = jnp.zeros((_NGP, _NA * (_NC + 5)), jnp.float32)
        for r in range(8):
            m = (rw == r).astype(jnp.float32)
            rows = rows + raw_scr[:, r, :] * m
        total = _sparse_math(stot_scr[0, 0], gtsr_ref[...], gtsc_ref[...],
                             rows)
        out_ref[...] = jnp.broadcast_to(total, (1, 1))


@jax.jit
def kernel(out, gts, size):
    del size
    f32 = jnp.float32
    gts_t = jnp.pad(gts.astype(f32).T, ((0, 0), (0, _NGP - _NGT)))
    gts_t_flat = gts_t.reshape(-1)
    gts_t8 = jnp.pad(gts_t, ((0, 1), (0, 0)))
    gts_p = jnp.pad(gts.astype(f32), ((0, _NGP - _NGT), (0, 1)))

    assign = pl.kernel(
        _sc_assign_body,
        out_type=jax.ShapeDtypeStruct((3 * _NGP,), jnp.int32),
        mesh=plsc.VectorSubcoreMesh(core_axis_name="c", subcore_axis_name="s"),
        compiler_params=pltpu.CompilerParams(needs_layout_passes=False),
        scratch_types=[
            pltpu.VMEM((7 * _NGP,), f32),
            pltpu.VMEM((48,), jnp.int32),
        ],
    )(gts_t_flat)

    nch = _NA * (_NC + 5)
    dview = out.transpose(0, 2, 3, 1).reshape(_NB * _NH, _NWG, nch)
    gview = out.transpose(0, 2, 3, 1).reshape(_NCELLBJ // 8, 8, nch)

    total = pl.pallas_call(
        _tc_main_body,
        grid_spec=pltpu.PrefetchScalarGridSpec(
            num_scalar_prefetch=1,
            grid=(_NSTEP,),
            in_specs=[
                pl.BlockSpec((_DROWS, _NWG, nch), lambda i, s: (i, 0, 0)),
                pl.BlockSpec(memory_space=pl.MemorySpace.ANY),
                pl.BlockSpec((8, _NGP), lambda i, s: (0, 0)),
                pl.BlockSpec((_NGP, 8), lambda i, s: (0, 0)),
            ],
            out_specs=pl.BlockSpec((1, 1), lambda i, s: (0, 0)),
            scratch_shapes=[
                pltpu.VMEM((_NGP, 8, nch), f32),
                pltpu.SMEM((1, 1), f32),
                pltpu.SemaphoreType.DMA,
            ],
        ),
        out_shape=jax.ShapeDtypeStruct((1, 1), f32),
    )(assign, dview, gview, gts_t8, gts_p)
    return total[0, 0]


# speedup vs baseline: 1.0094x; 1.0010x over previous
import jax
import jax.numpy as jnp
from jax import lax
from jax.experimental import pallas as pl
from jax.experimental.pallas import tpu as pltpu
from jax.experimental.pallas import tpu_sc as plsc

_NB, _NA, _NC, _NH, _NWG = 8, 3, 80, 64, 64
_AW = (0.05, 0.1, 0.2)
_AH = (0.08, 0.15, 0.3)
_NGT = 200
_NGP = 224
_NWORK = 14
_NCELL = _NB * _NA * _NH * _NWG
_NCELLBJ = _NB * _NH * _NWG
_EPS = 1e-7
_DROWS = 64
_NSTEP = _NB * _NH // _DROWS
_GPB = _NGP // _NSTEP


def _iou_anchor(w, h, aw, ah):
    inter = jnp.minimum(w, aw) * jnp.minimum(h, ah)
    union = w * h + aw * ah - inter + 1e-16
    return inter / union


def _best_anchor(w, h):
    i0 = _iou_anchor(w, h, _AW[0], _AH[0])
    i1 = _iou_anchor(w, h, _AW[1], _AH[1])
    i2 = _iou_anchor(w, h, _AW[2], _AH[2])
    best = jnp.where(i1 > i0, 1, 0)
    best = jnp.where(i2 > jnp.maximum(i0, i1), 2, best)
    return best, (i0, i1, i2)


def _sigmoid(x):
    return 1.0 / (1.0 + jnp.exp(-x))


def _sc_assign_body(gts_hbm, res_hbm, gts_v, asn_v):
    wid = lax.axis_index("s") * 2 + lax.axis_index("c")

    @pl.when(wid < _NWORK)
    def _():
        pltpu.sync_copy(gts_hbm, gts_v)
        base = wid * 16

        def fld(f):
            return gts_v[pl.ds(f * _NGP + base, 16)]

        bt, xf, yf, wf, hf = fld(0), fld(2), fld(3), fld(4), fld(5)
        best, _ = _best_anchor(wf, hf)
        bi = bt.astype(jnp.int32)
        gi = (xf * float(_NWG)).astype(jnp.int32)
        gj = (yf * float(_NH)).astype(jnp.int32)
        cell = (bi * _NH + gj) * _NWG + gi
        asn_v[pl.ds(0, 16)] = cell // 8
        asn_v[pl.ds(16, 16)] = cell - (cell // 8) * 8
        asn_v[pl.ds(32, 16)] = best
        pltpu.sync_copy(asn_v.at[pl.ds(0, 16)],
                        res_hbm.at[pl.ds(base, 16)])
        pltpu.sync_copy(asn_v.at[pl.ds(16, 16)],
                        res_hbm.at[pl.ds(_NGP + base, 16)])
        pltpu.sync_copy(asn_v.at[pl.ds(32, 16)],
                        res_hbm.at[pl.ds(2 * _NGP + base, 16)])


def _sparse_math(s_total, gtsr, gtsc, rows):
    f32 = jnp.float32

    def meta(fields):
        b, lab, x, y, w, h = fields
        best, ious = _best_anchor(w, h)
        bi = b.astype(jnp.int32)
        gi = (x * float(_NWG)).astype(jnp.int32)
        gj = (y * float(_NH)).astype(jnp.int32)
        cell = ((bi * _NA + best) * _NH + gj) * _NWG + gi
        anchor_cells = [((bi * _NA + a) * _NH + gj) * _NWG + gi
                        for a in range(_NA)]
        return best, cell, anchor_cells, ious, lab.astype(jnp.int32)

    col = [gtsc[:, k:k + 1] for k in range(6)]
    row = [gtsr[k:k + 1, :] for k in range(6)]
    best_c, cell_c, acells_c, ious_c, lab_c = meta(col)
    best_r, cell_r, acells_r, ious_r, lab_r = meta(row)

    sel = jnp.where(best_c == 0, rows[:, 0:85],
                    jnp.where(best_c == 1, rows[:, 85:170], rows[:, 170:255]))

    gidx_c = lax.broadcasted_iota(jnp.int32, (_NGP, 1), 0)
    gidx_r = lax.broadcasted_iota(jnp.int32, (1, _NGP), 1)
    valid_c = gidx_c < _NGT
    valid_r = gidx_r < _NGT

    eq = cell_c == cell_r
    later = gidx_r > gidx_c
    winner = valid_c & ~jnp.any(eq & later & valid_r, axis=1, keepdims=True)
    eql = (cell_c * _NC + lab_c) == (cell_r * _NC + lab_r)
    earlier = gidx_r < gidx_c
    labrep = valid_c & ~jnp.any(eql & earlier & valid_r, axis=1, keepdims=True)

    k3_c = jnp.concatenate(acells_c, axis=0)
    k3_r = jnp.concatenate(acells_r, axis=1)
    v3_c = jnp.concatenate(
        [(valid_c & ((best_c == a) | (ious_c[a] > 0.5))).astype(jnp.int32)
         for a in range(_NA)], axis=0) != 0
    v3_r = jnp.concatenate(
        [(valid_r & ((best_r == a) | (ious_r[a] > 0.5))).astype(jnp.int32)
         for a in range(_NA)], axis=1) != 0
    i3_c = lax.broadcasted_iota(jnp.int32, (_NA * _NGP, 1), 0)
    i3_r = lax.broadcasted_iota(jnp.int32, (1, _NA * _NGP), 1)
    first3 = v3_c & ~jnp.any((k3_c == k3_r) & (i3_r < i3_c) & v3_r,
                             axis=1, keepdims=True)
    conf3 = jnp.concatenate(
        [rows[:, a * 85 + 4:a * 85 + 5] for a in range(_NA)],
        axis=0)
    p3 = jnp.clip(_sigmoid(conf3), _EPS, 1.0 - _EPS)
    first3f = first3.astype(f32)
    s_nn = jnp.sum(first3f * (-jnp.log(1.0 - p3)))
    n_nn = jnp.sum(first3f)

    winf = winner.astype(f32)
    nm = jnp.sum(winf)
    n_noobj = float(_NCELL) - n_nn
    loss_conf_noobj = (s_total - s_nn) / jnp.maximum(n_noobj, 1.0)

    pobj = jnp.clip(_sigmoid(sel[:, 4:5]), _EPS, 1.0 - _EPS)
    objsum = jnp.sum(winf * (-jnp.log(pobj)))
    loss_conf_obj = objsum / jnp.maximum(nm, 1.0)
    loss_conf = 0.5 * loss_conf_noobj + 1.0 * loss_conf_obj

    xs = _sigmoid(sel[:, 0:1])
    ys = _sigmoid(sel[:, 1:2])
    ws = sel[:, 2:3]
    hs = sel[:, 3:4]
    aw = jnp.where(best_c == 0, _AW[0], jnp.where(best_c == 1, _AW[1], _AW[2]))
    ah = jnp.where(best_c == 0, _AH[0], jnp.where(best_c == 1, _AH[1], _AH[2]))
    tx = col[2] - jnp.floor(col[2])
    ty = col[3] - jnp.floor(col[3])
    reg = ((xs - tx) ** 2 + (ys - ty) ** 2
           + (ws - jnp.log(col[4] / aw + 1e-16)) ** 2
           + (hs - jnp.log(col[5] / ah + 1e-16)) ** 2)
    regsum = jnp.sum(winf * reg)
    loss_reg = regsum / jnp.maximum(nm, 1.0)

    logits = sel[:, 5:85]
    mx = jnp.max(logits, axis=1, keepdims=True)
    e = jnp.exp(logits - mx)
    sm = jnp.clip(e / jnp.sum(e, axis=1, keepdims=True), _EPS, 1.0 - _EPS)
    base_cls = jnp.sum(-jnp.log(1.0 - sm), axis=1, keepdims=True)
    onehot = (lax.broadcasted_iota(jnp.int32, (_NGP, _NC), 1)
              == lab_c).astype(f32)
    lterm = jnp.sum(onehot * (-jnp.log(sm) + jnp.log(1.0 - sm)),
                    axis=1, keepdims=True)
    clssum = jnp.sum(winf * base_cls) + jnp.sum(labrep.astype(f32) * lterm)
    loss_cls = clssum / jnp.maximum(float(_NC) * nm, 1.0)

    return nm * 1.0 * loss_reg + loss_conf + 1.0 * loss_cls * nm


def _tc_main_body(s_ref, dense_ref, gview_ref, gtsr_ref, gtsc_ref, out_ref,
                  raw_scr, stot_scr, sem):
    i = pl.program_id(0)

    def issue(k, _):
        g = i * _GPB + k
        tr = s_ref[g]
        pltpu.make_async_copy(gview_ref.at[pl.ds(tr, 1)],
                              raw_scr.at[pl.ds(g, 1)], sem).start()
        return 0

    lax.fori_loop(0, _GPB, issue, 0)

    s = jnp.float32(0.0)
    for a in range(_NA):
        c = a * (_NC + 5) + 4
        x = dense_ref[:, :, c]
        p = jnp.clip(_sigmoid(x), _EPS, 1.0 - _EPS)
        s = s + jnp.sum(-jnp.log(1.0 - p))

    @pl.when(i == 0)
    def _():
        stot_scr[0, 0] = jnp.float32(0.0)

    stot_scr[0, 0] += s

    @pl.when(i == _NSTEP - 1)
    def _():
        pltpu.make_async_copy(gview_ref.at[pl.ds(0, _NGP)], raw_scr,
                              sem).wait()
        xc = gtsc_ref[:, 2:3]
        yc = gtsc_ref[:, 3:4]
        bc = gtsc_ref[:, 0:1].astype(jnp.int32)
        gi = (xc * float(_NWG)).astype(jnp.int32)
        gj = (yc * float(_NH)).astype(jnp.int32)
        cell = (bc * _NH + gj) * _NWG + gi
        rw = cell - (cell // 8) * 8
        rows = jnp.zeros((_NGP, _NA * (_NC + 5)), jnp.float32)
        for r in range(8):
            m = (rw == r).astype(jnp.float32)
            rows = rows + raw_scr[:, r, :] * m
        total = _sparse_math(stot_scr[0, 0], gtsr_ref[...], gtsc_ref[...],
                             rows)
        out_ref[...] = jnp.broadcast_to(total, (1, 1))


@jax.jit
def kernel(out, gts, size):
    del size
    f32 = jnp.float32
    gts_t = jnp.pad(gts.astype(f32).T, ((0, 0), (0, _NGP - _NGT)))
    gts_t_flat = gts_t.reshape(-1)
    gts_t8 = jnp.pad(gts_t, ((0, 1), (0, 0)))
    gts_p = jnp.pad(gts.astype(f32), ((0, _NGP - _NGT), (0, 1)))

    assign = pl.kernel(
        _sc_assign_body,
        out_type=jax.ShapeDtypeStruct((3 * _NGP,), jnp.int32),
        mesh=plsc.VectorSubcoreMesh(core_axis_name="c", subcore_axis_name="s"),
        compiler_params=pltpu.CompilerParams(needs_layout_passes=False),
        scratch_types=[
            pltpu.VMEM((7 * _NGP,), f32),
            pltpu.VMEM((48,), jnp.int32),
        ],
    )(gts_t_flat)

    nch = _NA * (_NC + 5)
    dview = out.transpose(0, 2, 3, 1).reshape(_NB * _NH, _NWG, nch)
    gview = out.transpose(0, 2, 3, 1).reshape(_NCELLBJ // 8, 8, nch)

    total = pl.pallas_call(
        _tc_main_body,
        grid_spec=pltpu.PrefetchScalarGridSpec(
            num_scalar_prefetch=1,
            grid=(_NSTEP,),
            in_specs=[
                pl.BlockSpec((_DROWS, _NWG, nch), lambda i, s: (i, 0, 0)),
                pl.BlockSpec(memory_space=pl.MemorySpace.ANY),
                pl.BlockSpec((8, _NGP), lambda i, s: (0, 0)),
                pl.BlockSpec((_NGP, 8), lambda i, s: (0, 0)),
            ],
            out_specs=pl.BlockSpec((1, 1), lambda i, s: (0, 0)),
            scratch_shapes=[
                pltpu.VMEM((_NGP, 8, nch), f32),
                pltpu.SMEM((1, 1), f32),
                pltpu.SemaphoreType.DMA,
            ],
        ),
        out_shape=jax.ShapeDtypeStruct((1, 1), f32),
    )(assign, dview, gview, gts_t8, gts_p)
    return total[0, 0]
